# Initial kernel scaffold; baseline (speedup 1.0000x reference)
import jax
import jax.numpy as jnp
from jax.experimental import pallas as pl


def kernel(z, clusters):
    raise NotImplementedError("write your pallas kernel here")


# speedup vs baseline: 1.0900x; 1.0900x over previous
import functools

import jax
import jax.numpy as jnp
from jax.experimental import pallas as pl

_LAMBDA2 = 1.0
_K = 1024
_D = 64
_BM = 512


def _cluster_kernel(z_ref, c_ref, z2_ref, c2_ref, q_ref, idx_ref, loss_ref):
    i = pl.program_id(0)
    z = z_ref[...]
    c = c_ref[...]
    zc = jax.lax.dot_general(
        z, c, (((1,), (1,)), ((), ())),
        preferred_element_type=jnp.float32)
    dist = (z2_ref[...] + c2_ref[...]) - 2.0 * zc
    minv = jnp.min(dist, axis=1, keepdims=True)
    iota = jax.lax.broadcasted_iota(jnp.int32, (_BM, _K), 1)
    idx = jnp.min(jnp.where(dist == minv, iota, _K), axis=1)
    idx_ref[0, 0, :] = idx
    onehot = (iota == idx[:, None]).astype(jnp.float32)
    q = jax.lax.dot_general(
        onehot, c, (((1,), (0,)), ((), ())),
        preferred_element_type=jnp.float32)
    q_ref[...] = q
    diff = z - q
    part = jnp.reshape(_LAMBDA2 * 0.5 * jnp.sum(diff * diff), (1, 1))

    @pl.when(i == 0)
    def _():
        loss_ref[...] = jnp.zeros((1, 1), jnp.float32)

    loss_ref[...] += part


@functools.partial(jax.jit, static_argnames=())
def kernel(z, clusters):
    B, N, D = z.shape
    M = B * N
    nb = M // _BM
    zf = z.reshape(M, D)
    z2 = jnp.sum(zf * zf, axis=1, keepdims=True)
    c2 = jnp.sum(clusters * clusters, axis=1)[None, :]

    q, idx3, loss = pl.pallas_call(
        _cluster_kernel,
        grid=(nb,),
        in_specs=[
            pl.BlockSpec((_BM, D), lambda i: (i, 0)),
            pl.BlockSpec((_K, D), lambda i: (0, 0)),
            pl.BlockSpec((_BM, 1), lambda i: (i, 0)),
            pl.BlockSpec((1, _K), lambda i: (0, 0)),
        ],
        out_specs=[
            pl.BlockSpec((_BM, D), lambda i: (i, 0)),
            pl.BlockSpec((1, 1, _BM), lambda i: (i, 0, 0)),
            pl.BlockSpec((1, 1), lambda i: (0, 0)),
        ],
        out_shape=[
            jax.ShapeDtypeStruct((M, D), jnp.float32),
            jax.ShapeDtypeStruct((nb, 1, _BM), jnp.int32),
            jax.ShapeDtypeStruct((1, 1), jnp.float32),
        ],
    )(zf, clusters, z2, c2)

    return q.reshape(B, N, D), idx3.reshape(B, N), loss.reshape(())


# speedup vs baseline: 1.1214x; 1.0289x over previous
import functools

import jax
import jax.numpy as jnp
from jax.experimental import pallas as pl

_LAMBDA2 = 1.0
_K = 1024
_D = 64
_BM = 512


def _cluster_kernel(z_ref, c_ref, z2_ref, c2_ref, rev_ref, q_ref, idx_ref,
                    loss_ref):
    i = pl.program_id(0)
    z = z_ref[...]
    c = c_ref[...]
    zc = jax.lax.dot_general(
        z, c, (((1,), (1,)), ((), ())),
        preferred_element_type=jnp.float32)
    dist = (z2_ref[...] + c2_ref[...]) - 2.0 * zc
    minv = jnp.min(dist, axis=1, keepdims=True)
    iota = jax.lax.broadcasted_iota(jnp.int32, (_BM, _K), 1)
    w = jnp.where(dist == minv, rev_ref[...], 0.0)
    idx = (float(_K) - jnp.max(w, axis=1)).astype(jnp.int32)
    idx_ref[0, 0, :] = idx
    onehot = (iota == idx[:, None]).astype(jnp.float32)
    q = jax.lax.dot_general(
        onehot, c, (((1,), (0,)), ((), ())),
        preferred_element_type=jnp.float32)
    q_ref[...] = q
    diff = z - q
    part = jnp.reshape(_LAMBDA2 * 0.5 * jnp.sum(diff * diff), (1, 1))

    @pl.when(i == 0)
    def _():
        loss_ref[...] = jnp.zeros((1, 1), jnp.float32)

    loss_ref[...] += part


@functools.partial(jax.jit, static_argnames=())
def kernel(z, clusters):
    B, N, D = z.shape
    M = B * N
    nb = M // _BM
    zf = z.reshape(M, D)
    z2 = jnp.sum(zf * zf, axis=1, keepdims=True)
    c2 = jnp.sum(clusters * clusters, axis=1)[None, :]
    rev = (float(_K) - jnp.arange(_K, dtype=jnp.float32))[None, :]

    q, idx3, loss = pl.pallas_call(
        _cluster_kernel,
        grid=(nb,),
        in_specs=[
            pl.BlockSpec((_BM, D), lambda i: (i, 0)),
            pl.BlockSpec((_K, D), lambda i: (0, 0)),
            pl.BlockSpec((_BM, 1), lambda i: (i, 0)),
            pl.BlockSpec((1, _K), lambda i: (0, 0)),
            pl.BlockSpec((1, _K), lambda i: (0, 0)),
        ],
        out_specs=[
            pl.BlockSpec((_BM, D), lambda i: (i, 0)),
            pl.BlockSpec((1, 1, _BM), lambda i: (i, 0, 0)),
            pl.BlockSpec((1, 1), lambda i: (0, 0)),
        ],
        out_shape=[
            jax.ShapeDtypeStruct((M, D), jnp.float32),
            jax.ShapeDtypeStruct((nb, 1, _BM), jnp.int32),
            jax.ShapeDtypeStruct((1, 1), jnp.float32),
        ],
    )(zf, clusters, z2, c2, rev)

    return q.reshape(B, N, D), idx3.reshape(B, N), loss.reshape(())


# speedup vs baseline: 1.1943x; 1.0650x over previous
import functools

import jax
import jax.numpy as jnp
from jax.experimental import pallas as pl

_LAMBDA2 = 1.0
_K = 1024
_D = 64
_BM = 1024


def _cluster_kernel(z_ref, c_ref, z2_ref, c2_ref, rev_ref, q_ref, idx_ref,
                    loss_ref):
    i = pl.program_id(0)
    z = z_ref[...]
    c = c_ref[...]
    zc = jax.lax.dot_general(
        z, c, (((1,), (1,)), ((), ())),
        preferred_element_type=jnp.float32)
    dist = (z2_ref[...] + c2_ref[...]) - 2.0 * zc
    minv = jnp.min(dist, axis=1, keepdims=True)
    iota = jax.lax.broadcasted_iota(jnp.int32, (_BM, _K), 1)
    w = jnp.where(dist == minv, rev_ref[...], 0.0)
    idx = (float(_K) - jnp.max(w, axis=1)).astype(jnp.int32)
    idx_ref[0, 0, :] = idx
    onehot = (iota == idx[:, None]).astype(jnp.float32)
    q = jax.lax.dot_general(
        onehot, c, (((1,), (0,)), ((), ())),
        preferred_element_type=jnp.float32)
    q_ref[...] = q
    diff = z - q
    part = jnp.reshape(_LAMBDA2 * 0.5 * jnp.sum(diff * diff), (1, 1))

    @pl.when(i == 0)
    def _():
        loss_ref[...] = jnp.zeros((1, 1), jnp.float32)

    loss_ref[...] += part


@functools.partial(jax.jit, static_argnames=())
def kernel(z, clusters):
    B, N, D = z.shape
    M = B * N
    nb = M // _BM
    zf = z.reshape(M, D)
    z2 = jnp.sum(zf * zf, axis=1, keepdims=True)
    c2 = jnp.sum(clusters * clusters, axis=1)[None, :]
    rev = (float(_K) - jnp.arange(_K, dtype=jnp.float32))[None, :]

    q, idx3, loss = pl.pallas_call(
        _cluster_kernel,
        grid=(nb,),
        in_specs=[
            pl.BlockSpec((_BM, D), lambda i: (i, 0)),
            pl.BlockSpec((_K, D), lambda i: (0, 0)),
            pl.BlockSpec((_BM, 1), lambda i: (i, 0)),
            pl.BlockSpec((1, _K), lambda i: (0, 0)),
            pl.BlockSpec((1, _K), lambda i: (0, 0)),
        ],
        out_specs=[
            pl.BlockSpec((_BM, D), lambda i: (i, 0)),
            pl.BlockSpec((1, 1, _BM), lambda i: (i, 0, 0)),
            pl.BlockSpec((1, 1), lambda i: (0, 0)),
        ],
        out_shape=[
            jax.ShapeDtypeStruct((M, D), jnp.float32),
            jax.ShapeDtypeStruct((nb, 1, _BM), jnp.int32),
            jax.ShapeDtypeStruct((1, 1), jnp.float32),
        ],
    )(zf, clusters, z2, c2, rev)

    return q.reshape(B, N, D), idx3.reshape(B, N), loss.reshape(())


# speedup vs baseline: 1.1995x; 1.0043x over previous
import functools

import jax
import jax.numpy as jnp
from jax.experimental import pallas as pl

_LAMBDA2 = 1.0
_K = 1024
_D = 64
_BM = 1152


def _cluster_kernel(z_ref, c_ref, z2_ref, c2_ref, rev_ref, q_ref, idx_ref,
                    loss_ref):
    i = pl.program_id(0)
    z = z_ref[...]
    c = c_ref[...]
    zc = jax.lax.dot_general(
        z, c, (((1,), (1,)), ((), ())),
        preferred_element_type=jnp.float32)
    dist = (z2_ref[...] + c2_ref[...]) - 2.0 * zc
    minv = jnp.min(dist, axis=1, keepdims=True)
    iota = jax.lax.broadcasted_iota(jnp.int32, (_BM, _K), 1)
    w = jnp.where(dist == minv, rev_ref[...], 0.0)
    idx = (float(_K) - jnp.max(w, axis=1)).astype(jnp.int32)
    idx_ref[0, 0, :] = idx
    onehot = (iota == idx[:, None]).astype(jnp.float32)
    q = jax.lax.dot_general(
        onehot, c, (((1,), (0,)), ((), ())),
        preferred_element_type=jnp.float32)
    q_ref[...] = q
    diff = z - q
    part = jnp.reshape(_LAMBDA2 * 0.5 * jnp.sum(diff * diff), (1, 1))

    @pl.when(i == 0)
    def _():
        loss_ref[...] = jnp.zeros((1, 1), jnp.float32)

    loss_ref[...] += part


@functools.partial(jax.jit, static_argnames=())
def kernel(z, clusters):
    B, N, D = z.shape
    M = B * N
    nb = M // _BM
    zf = z.reshape(M, D)
    z2 = jnp.sum(zf * zf, axis=1, keepdims=True)
    c2 = jnp.sum(clusters * clusters, axis=1)[None, :]
    rev = (float(_K) - jnp.arange(_K, dtype=jnp.float32))[None, :]

    q, idx3, loss = pl.pallas_call(
        _cluster_kernel,
        grid=(nb,),
        in_specs=[
            pl.BlockSpec((_BM, D), lambda i: (i, 0)),
            pl.BlockSpec((_K, D), lambda i: (0, 0)),
            pl.BlockSpec((_BM, 1), lambda i: (i, 0)),
            pl.BlockSpec((1, _K), lambda i: (0, 0)),
            pl.BlockSpec((1, _K), lambda i: (0, 0)),
        ],
        out_specs=[
            pl.BlockSpec((_BM, D), lambda i: (i, 0)),
            pl.BlockSpec((1, 1, _BM), lambda i: (i, 0, 0)),
            pl.BlockSpec((1, 1), lambda i: (0, 0)),
        ],
        out_shape=[
            jax.ShapeDtypeStruct((M, D), jnp.float32),
            jax.ShapeDtypeStruct((nb, 1, _BM), jnp.int32),
            jax.ShapeDtypeStruct((1, 1), jnp.float32),
        ],
    )(zf, clusters, z2, c2, rev)

    return q.reshape(B, N, D), idx3.reshape(B, N), loss.reshape(())


# speedup vs baseline: 1.2195x; 1.0166x over previous
import functools

import jax
import jax.numpy as jnp
from jax.experimental import pallas as pl

_LAMBDA2 = 1.0
_K = 1024
_D = 64
_BM = 2304


def _cluster_kernel(z_ref, c_ref, z2_ref, c2_ref, rev_ref, q_ref, idx_ref,
                    loss_ref):
    i = pl.program_id(0)
    z = z_ref[...]
    c = c_ref[...]
    zc = jax.lax.dot_general(
        z, c, (((1,), (1,)), ((), ())),
        preferred_element_type=jnp.float32)
    dist = (z2_ref[...] + c2_ref[...]) - 2.0 * zc
    minv = jnp.min(dist, axis=1, keepdims=True)
    iota = jax.lax.broadcasted_iota(jnp.int32, (_BM, _K), 1)
    w = jnp.where(dist == minv, rev_ref[...], 0.0)
    idx = (float(_K) - jnp.max(w, axis=1)).astype(jnp.int32)
    idx_ref[0, 0, :] = idx
    onehot = (iota == idx[:, None]).astype(jnp.float32)
    q = jax.lax.dot_general(
        onehot, c, (((1,), (0,)), ((), ())),
        preferred_element_type=jnp.float32)
    q_ref[...] = q
    diff = z - q
    part = jnp.reshape(_LAMBDA2 * 0.5 * jnp.sum(diff * diff), (1, 1))

    @pl.when(i == 0)
    def _():
        loss_ref[...] = jnp.zeros((1, 1), jnp.float32)

    loss_ref[...] += part


@functools.partial(jax.jit, static_argnames=())
def kernel(z, clusters):
    B, N, D = z.shape
    M = B * N
    nb = M // _BM
    zf = z.reshape(M, D)
    z2 = jnp.sum(zf * zf, axis=1, keepdims=True)
    c2 = jnp.sum(clusters * clusters, axis=1)[None, :]
    rev = (float(_K) - jnp.arange(_K, dtype=jnp.float32))[None, :]

    q, idx3, loss = pl.pallas_call(
        _cluster_kernel,
        grid=(nb,),
        in_specs=[
            pl.BlockSpec((_BM, D), lambda i: (i, 0)),
            pl.BlockSpec((_K, D), lambda i: (0, 0)),
            pl.BlockSpec((_BM, 1), lambda i: (i, 0)),
            pl.BlockSpec((1, _K), lambda i: (0, 0)),
            pl.BlockSpec((1, _K), lambda i: (0, 0)),
        ],
        out_specs=[
            pl.BlockSpec((_BM, D), lambda i: (i, 0)),
            pl.BlockSpec((1, 1, _BM), lambda i: (i, 0, 0)),
            pl.BlockSpec((1, 1), lambda i: (0, 0)),
        ],
        out_shape=[
            jax.ShapeDtypeStruct((M, D), jnp.float32),
            jax.ShapeDtypeStruct((nb, 1, _BM), jnp.int32),
            jax.ShapeDtypeStruct((1, 1), jnp.float32),
        ],
    )(zf, clusters, z2, c2, rev)

    return q.reshape(B, N, D), idx3.reshape(B, N), loss.reshape(())
